# baseline (speedup 1.0000x reference)
import functools

import jax
import jax.numpy as jnp
from jax.experimental import pallas as pl
from jax.experimental.pallas import tpu as pltpu

_LANES = 128


def _pool_body(x_ref, o_ref, *, n_full, tail, inv_l):
    if n_full == 0:
        s = jnp.sum(x_ref[...].astype(jnp.float32), axis=-1, keepdims=True)
    else:
        acc = x_ref[:, 0:_LANES].astype(jnp.float32)
        for c in range(1, n_full):
            acc = acc + x_ref[:, c * _LANES:(c + 1) * _LANES].astype(jnp.float32)
        s = jnp.sum(acc, axis=-1, keepdims=True)
        if tail:
            t = x_ref[:, n_full * _LANES:].astype(jnp.float32)
            s = s + jnp.sum(t, axis=-1, keepdims=True)
    o_ref[...] = (s * jnp.float32(inv_l)).astype(o_ref.dtype)


def _pick_tb(B, L, itemsize, target_bytes=16 << 20):
    row_bytes = L * itemsize
    tb = max(8, min(2048, (target_bytes // row_bytes) // 8 * 8))
    while tb > 8 and -(-B // tb) < 2:
        tb //= 2
    return tb


def kernel(x):
    shape = x.shape
    L = shape[-1]
    lead = shape[:-1]
    B = 1
    for d in lead:
        B *= d
    x2 = x.reshape(B, L)

    itemsize = jnp.dtype(x.dtype).itemsize
    TB = _pick_tb(B, L, itemsize)
    grid_b = -(-B // TB)

    n_full = L // _LANES
    tail = L % _LANES != 0

    in_block = TB * L * itemsize
    vlim = int(min(max(4 * in_block + (4 << 20), 16 << 20), 48 << 20))
    out = pl.pallas_call(
        functools.partial(_pool_body, n_full=n_full, tail=tail, inv_l=1.0 / L),
        out_shape=jax.ShapeDtypeStruct((B, 1), x.dtype),
        grid=(grid_b,),
        in_specs=[pl.BlockSpec((TB, L), lambda b: (b, 0))],
        out_specs=pl.BlockSpec((TB, 1), lambda b: (b, 0)),
        compiler_params=pltpu.CompilerParams(
            dimension_semantics=("parallel",),
            vmem_limit_bytes=vlim),
    )(x2)

    return out.reshape(lead)


# speedup vs baseline: 1.0933x; 1.0933x over previous
import functools

import jax
import jax.numpy as jnp
from jax.experimental import pallas as pl
from jax.experimental.pallas import tpu as pltpu

_LANES = 128


def _row_mean(x_ref, n_full, tail, inv_l):
    if n_full == 0:
        s = jnp.sum(x_ref[...].astype(jnp.float32), axis=-1, keepdims=True)
    else:
        acc = x_ref[:, 0:_LANES].astype(jnp.float32)
        for c in range(1, n_full):
            acc = acc + x_ref[:, c * _LANES:(c + 1) * _LANES].astype(jnp.float32)
        s = jnp.sum(acc, axis=-1, keepdims=True)
        if tail:
            t = x_ref[:, n_full * _LANES:].astype(jnp.float32)
            s = s + jnp.sum(t, axis=-1, keepdims=True)
    return s * jnp.float32(inv_l)


def _pool_body2(x0_ref, x1_ref, o_ref, *, n_full, tail, inv_l, tb):
    o_ref[0:tb, :] = _row_mean(x0_ref, n_full, tail, inv_l).astype(o_ref.dtype)
    o_ref[tb:2 * tb, :] = _row_mean(x1_ref, n_full, tail, inv_l).astype(o_ref.dtype)


def _pool_body1(x_ref, o_ref, *, n_full, tail, inv_l):
    o_ref[...] = _row_mean(x_ref, n_full, tail, inv_l).astype(o_ref.dtype)


def _pick_tb(B, L, itemsize, target_bytes):
    row_bytes = L * itemsize
    tb = max(8, min(2048, (target_bytes // row_bytes) // 8 * 8))
    while tb > 8 and -(-B // tb) < 2:
        tb //= 2
    return tb


def kernel(x):
    shape = x.shape
    L = shape[-1]
    lead = shape[:-1]
    B = 1
    for d in lead:
        B *= d
    x2 = x.reshape(B, L)

    itemsize = jnp.dtype(x.dtype).itemsize
    n_full = L // _LANES
    tail = L % _LANES != 0
    inv_l = 1.0 / L

    TB = _pick_tb(B, L, itemsize, target_bytes=4 << 20)
    two_stream = B % (2 * TB) == 0 and B // (2 * TB) >= 2
    if two_stream:
        grid_b = B // (2 * TB)
        in_block = 2 * TB * L * itemsize
        vlim = int(min(max(4 * in_block + (4 << 20), 16 << 20), 48 << 20))
        out = pl.pallas_call(
            functools.partial(_pool_body2, n_full=n_full, tail=tail,
                              inv_l=inv_l, tb=TB),
            out_shape=jax.ShapeDtypeStruct((B, 1), x.dtype),
            grid=(grid_b,),
            in_specs=[pl.BlockSpec((TB, L), lambda b: (2 * b, 0)),
                      pl.BlockSpec((TB, L), lambda b: (2 * b + 1, 0))],
            out_specs=pl.BlockSpec((2 * TB, 1), lambda b: (b, 0)),
            compiler_params=pltpu.CompilerParams(
                dimension_semantics=("parallel",),
                vmem_limit_bytes=vlim),
        )(x2, x2)
    else:
        TB = _pick_tb(B, L, itemsize, target_bytes=8 << 20)
        grid_b = -(-B // TB)
        in_block = TB * L * itemsize
        vlim = int(min(max(4 * in_block + (4 << 20), 16 << 20), 48 << 20))
        out = pl.pallas_call(
            functools.partial(_pool_body1, n_full=n_full, tail=tail,
                              inv_l=inv_l),
            out_shape=jax.ShapeDtypeStruct((B, 1), x.dtype),
            grid=(grid_b,),
            in_specs=[pl.BlockSpec((TB, L), lambda b: (b, 0))],
            out_specs=pl.BlockSpec((TB, 1), lambda b: (b, 0)),
            compiler_params=pltpu.CompilerParams(
                dimension_semantics=("parallel",),
                vmem_limit_bytes=vlim),
        )(x2)

    return out.reshape(lead)
